# baseline (speedup 1.0000x reference)
import functools

import jax
import jax.numpy as jnp
from jax import lax
from jax.experimental import pallas as pl
from jax.experimental.pallas import tpu as pltpu
from jax.experimental.pallas import tpu_sc as plsc

D = 128
L = 16
NC = 2
NS = 16
NW = NC * NS
C = 80
B = 400


@functools.partial(jax.jit, static_argnums=(3,))
def _run(src_idx, dst_idx, z, E):
    e_per_w = E // NW
    n_chunks = e_per_w // C
    mesh = plsc.VectorSubcoreMesh(core_axis_name="c", subcore_axis_name="s")

    @functools.partial(
        pl.kernel,
        mesh=mesh,
        compiler_params=pltpu.CompilerParams(needs_layout_passes=False),
        out_type=jax.ShapeDtypeStruct((E,), jnp.float32),
        scratch_types=[
            pltpu.VMEM_SHARED((10000, D), jnp.float32),
            pltpu.VMEM((B,), jnp.int32),
            pltpu.VMEM((B,), jnp.int32),
            pltpu.VMEM((C, D), jnp.float32),
            pltpu.VMEM((C, D), jnp.float32),
            pltpu.VMEM((C, D), jnp.float32),
            pltpu.VMEM((C, D), jnp.float32),
            pltpu.VMEM((B,), jnp.float32),
            pltpu.SemaphoreType.DMA,
            pltpu.SemaphoreType.DMA,
        ],
    )
    def k(src_hbm, dst_hbm, z_hbm, out_hbm, z_sh, idx_s, idx_d, rs0, rd0,
          rs1, rd1, out_b, sem0, sem1):
        cid = lax.axis_index("c")
        sid = lax.axis_index("s")
        wid = sid * NC + cid
        base = wid * e_per_w

        n_rows = 624
        pltpu.sync_copy(z_hbm.at[pl.ds(sid * n_rows, n_rows)],
                        z_sh.at[pl.ds(sid * n_rows, n_rows)])

        @pl.when(sid == NS - 1)
        def _stage_tail():
            pltpu.sync_copy(z_hbm.at[pl.ds(NS * n_rows, 10000 - NS * n_rows)],
                            z_sh.at[pl.ds(NS * n_rows, 10000 - NS * n_rows)])

        plsc.subcore_barrier()

        bufs = ((rs0, rd0, sem0), (rs1, rd1, sem1))
        lane = lax.iota(jnp.int32, L)
        cpb = B // C

        def issue(c, b):
            rs, rd, sem = bufs[b]
            off = c * C
            pltpu.async_copy(z_sh.at[idx_s.at[pl.ds(off, C)]], rs, sem)
            pltpu.async_copy(z_sh.at[idx_d.at[pl.ds(off, C)]], rd, sem)

        def wait(b):
            rs, rd, sem = bufs[b]
            pltpu.make_async_copy(z_hbm.at[idx_s.at[pl.ds(0, C)]], rs,
                                  sem).wait()
            pltpu.make_async_copy(z_hbm.at[idx_d.at[pl.ds(0, C)]], rd,
                                  sem).wait()

        def compute(c, b):
            rs, rd, _ = bufs[b]
            off = c * C

            def group_body(g, carry2):
                rid = lane + g * L
                acc = jnp.zeros((L,), jnp.float32)
                for f in range(D):
                    col = jnp.full((L,), f, jnp.int32)
                    sv = plsc.load_gather(rs, [rid, col])
                    dv = plsc.load_gather(rd, [rid, col])
                    acc = acc + sv * dv
                sig = 1.0 / (1.0 + jnp.exp(-acc))
                out_b[pl.ds(off + g * L, L)] = sig
                return carry2

            lax.fori_loop(0, C // L, group_body, 0)

        def block_body(blk, carry):
            bbase = base + blk * B
            pltpu.sync_copy(src_hbm.at[pl.ds(bbase, B)], idx_s)
            pltpu.sync_copy(dst_hbm.at[pl.ds(bbase, B)], idx_d)
            issue(0, 0)

            def pair_body(t, carry2):
                c0 = t * 2
                issue(c0 + 1, 1)
                wait(0)
                compute(c0, 0)
                issue(c0 + 2, 0)
                wait(1)
                compute(c0 + 1, 1)
                return carry2

            lax.fori_loop(0, (cpb - 1) // 2, pair_body, 0)
            wait(0)
            compute(cpb - 1, 0)
            pltpu.sync_copy(out_b, out_hbm.at[pl.ds(bbase, B)])
            return carry

        lax.fori_loop(0, e_per_w // B, block_body, 0)

    return k(src_idx, dst_idx, z)


def kernel(z, edge_index):
    idx = edge_index.astype(jnp.int32)
    return _run(idx[0], idx[1], z, idx.shape[1])


# speedup vs baseline: 6.2578x; 6.2578x over previous
import functools

import jax
import jax.numpy as jnp
from jax import lax
from jax.experimental import pallas as pl
from jax.experimental.pallas import tpu as pltpu
from jax.experimental.pallas import tpu_sc as plsc

D = 128
L = 16
NC = 2
NS = 16
NW = NC * NS
C = 80
B = 400


@functools.partial(jax.jit, static_argnums=(3,))
def _run(src_idx, dst_idx, z, E):
    e_per_w = E // NW
    n_chunks = e_per_w // C
    mesh = plsc.VectorSubcoreMesh(core_axis_name="c", subcore_axis_name="s")

    @functools.partial(
        pl.kernel,
        mesh=mesh,
        compiler_params=pltpu.CompilerParams(needs_layout_passes=False),
        out_type=jax.ShapeDtypeStruct((E,), jnp.float32),
        scratch_types=[
            pltpu.VMEM_SHARED((10000, D), jnp.float32),
            pltpu.VMEM((B,), jnp.int32),
            pltpu.VMEM((B,), jnp.int32),
            pltpu.VMEM((C, D), jnp.float32),
            pltpu.VMEM((C, D), jnp.float32),
            pltpu.VMEM((C, D), jnp.float32),
            pltpu.VMEM((C, D), jnp.float32),
            pltpu.VMEM((B,), jnp.float32),
            pltpu.SemaphoreType.DMA,
            pltpu.SemaphoreType.DMA,
        ],
    )
    def k(src_hbm, dst_hbm, z_hbm, out_hbm, z_sh, idx_s, idx_d, rs0, rd0,
          rs1, rd1, out_b, sem0, sem1):
        cid = lax.axis_index("c")
        sid = lax.axis_index("s")
        wid = sid * NC + cid
        base = wid * e_per_w

        n_rows = 624
        pltpu.sync_copy(z_hbm.at[pl.ds(sid * n_rows, n_rows)],
                        z_sh.at[pl.ds(sid * n_rows, n_rows)])

        @pl.when(sid == NS - 1)
        def _stage_tail():
            pltpu.sync_copy(z_hbm.at[pl.ds(NS * n_rows, 10000 - NS * n_rows)],
                            z_sh.at[pl.ds(NS * n_rows, 10000 - NS * n_rows)])

        plsc.subcore_barrier()

        bufs = ((rs0, rd0, sem0), (rs1, rd1, sem1))
        lane = lax.iota(jnp.int32, L)
        cpb = B // C

        def issue(c, b):
            rs, rd, sem = bufs[b]
            off = c * C
            pltpu.async_copy(z_sh.at[idx_s.at[pl.ds(off, C)]], rs, sem)
            pltpu.async_copy(z_sh.at[idx_d.at[pl.ds(off, C)]], rd, sem)

        def wait(b):
            rs, rd, sem = bufs[b]
            pltpu.make_async_copy(z_hbm.at[idx_s.at[pl.ds(0, C)]], rs,
                                  sem).wait()
            pltpu.make_async_copy(z_hbm.at[idx_d.at[pl.ds(0, C)]], rd,
                                  sem).wait()

        def compute(c, b):
            rs, rd, _ = bufs[b]
            off = c * C

            def group_body(g, carry2):
                e0 = g * L

                def edge_body(j, res):
                    e = e0 + j
                    acc = jnp.zeros((L,), jnp.float32)
                    for kk in range(D // L):
                        sv = rs[e, pl.ds(kk * L, L)]
                        dv = rd[e, pl.ds(kk * L, L)]
                        acc = acc + sv * dv
                    dot = jnp.sum(acc)
                    return jnp.where(lane == j, dot, res)

                res = lax.fori_loop(0, L, edge_body,
                                    jnp.zeros((L,), jnp.float32), unroll=4)
                sig = 1.0 / (1.0 + jnp.exp(-res))
                out_b[pl.ds(off + g * L, L)] = sig
                return carry2

            lax.fori_loop(0, C // L, group_body, 0)

        def block_body(blk, carry):
            bbase = base + blk * B
            pltpu.sync_copy(src_hbm.at[pl.ds(bbase, B)], idx_s)
            pltpu.sync_copy(dst_hbm.at[pl.ds(bbase, B)], idx_d)
            issue(0, 0)

            def pair_body(t, carry2):
                c0 = t * 2
                issue(c0 + 1, 1)
                wait(0)
                compute(c0, 0)
                issue(c0 + 2, 0)
                wait(1)
                compute(c0 + 1, 1)
                return carry2

            lax.fori_loop(0, (cpb - 1) // 2, pair_body, 0)
            wait(0)
            compute(cpb - 1, 0)
            pltpu.sync_copy(out_b, out_hbm.at[pl.ds(bbase, B)])
            return carry

        lax.fori_loop(0, e_per_w // B, block_body, 0)

    return k(src_idx, dst_idx, z)


def kernel(z, edge_index):
    idx = edge_index.astype(jnp.int32)
    return _run(idx[0], idx[1], z, idx.shape[1])


# speedup vs baseline: 7.0841x; 1.1321x over previous
import functools

import jax
import jax.numpy as jnp
from jax import lax
from jax.experimental import pallas as pl
from jax.experimental.pallas import tpu as pltpu
from jax.experimental.pallas import tpu_sc as plsc

D = 128
L = 16
NC = 2
NS = 16
NW = NC * NS
C = 80
B = 2000


@functools.partial(jax.jit, static_argnums=(3,))
def _run(src_idx, dst_idx, z, E):
    e_per_w = E // NW
    n_chunks = e_per_w // C
    mesh = plsc.VectorSubcoreMesh(core_axis_name="c", subcore_axis_name="s")

    @functools.partial(
        pl.kernel,
        mesh=mesh,
        compiler_params=pltpu.CompilerParams(needs_layout_passes=False),
        out_type=jax.ShapeDtypeStruct((E,), jnp.float32),
        scratch_types=[
            pltpu.VMEM_SHARED((10000, D), jnp.float32),
            pltpu.VMEM((B,), jnp.int32),
            pltpu.VMEM((B,), jnp.int32),
            pltpu.VMEM((C, D), jnp.float32),
            pltpu.VMEM((C, D), jnp.float32),
            pltpu.VMEM((C, D), jnp.float32),
            pltpu.VMEM((C, D), jnp.float32),
            pltpu.VMEM((B,), jnp.float32),
            pltpu.SemaphoreType.DMA,
            pltpu.SemaphoreType.DMA,
        ],
    )
    def k(src_hbm, dst_hbm, z_hbm, out_hbm, z_sh, idx_s, idx_d, rs0, rd0,
          rs1, rd1, out_b, sem0, sem1):
        cid = lax.axis_index("c")
        sid = lax.axis_index("s")
        wid = sid * NC + cid
        base = wid * e_per_w

        n_rows = 624
        pltpu.sync_copy(z_hbm.at[pl.ds(sid * n_rows, n_rows)],
                        z_sh.at[pl.ds(sid * n_rows, n_rows)])

        @pl.when(sid == NS - 1)
        def _stage_tail():
            pltpu.sync_copy(z_hbm.at[pl.ds(NS * n_rows, 10000 - NS * n_rows)],
                            z_sh.at[pl.ds(NS * n_rows, 10000 - NS * n_rows)])

        plsc.subcore_barrier()

        bufs = ((rs0, rd0, sem0), (rs1, rd1, sem1))
        lane = lax.iota(jnp.int32, L)
        cpb = B // C

        def issue(c, b):
            rs, rd, sem = bufs[b]
            off = c * C
            pltpu.async_copy(z_sh.at[idx_s.at[pl.ds(off, C)]], rs, sem)
            pltpu.async_copy(z_sh.at[idx_d.at[pl.ds(off, C)]], rd, sem)

        def wait(b):
            rs, rd, sem = bufs[b]
            pltpu.make_async_copy(z_hbm.at[idx_s.at[pl.ds(0, C)]], rs,
                                  sem).wait()
            pltpu.make_async_copy(z_hbm.at[idx_d.at[pl.ds(0, C)]], rd,
                                  sem).wait()

        def compute(c, b):
            rs, rd, _ = bufs[b]
            off = c * C

            def group_body(g, carry2):
                e0 = g * L

                def edge_body(j, res):
                    e = e0 + j
                    acc = jnp.zeros((L,), jnp.float32)
                    for kk in range(D // L):
                        sv = rs[e, pl.ds(kk * L, L)]
                        dv = rd[e, pl.ds(kk * L, L)]
                        acc = acc + sv * dv
                    dot = jnp.sum(acc)
                    return jnp.where(lane == j, dot, res)

                res = lax.fori_loop(0, L, edge_body,
                                    jnp.zeros((L,), jnp.float32), unroll=8)
                sig = 1.0 / (1.0 + jnp.exp(-res))
                out_b[pl.ds(off + g * L, L)] = sig
                return carry2

            lax.fori_loop(0, C // L, group_body, 0)

        def block_body(blk, carry):
            bbase = base + blk * B
            pltpu.sync_copy(src_hbm.at[pl.ds(bbase, B)], idx_s)
            pltpu.sync_copy(dst_hbm.at[pl.ds(bbase, B)], idx_d)
            issue(0, 0)

            def pair_body(t, carry2):
                c0 = t * 2
                issue(c0 + 1, 1)
                wait(0)
                compute(c0, 0)
                issue(c0 + 2, 0)
                wait(1)
                compute(c0 + 1, 1)
                return carry2

            lax.fori_loop(0, (cpb - 1) // 2, pair_body, 0)
            wait(0)
            compute(cpb - 1, 0)
            pltpu.sync_copy(out_b, out_hbm.at[pl.ds(bbase, B)])
            return carry

        lax.fori_loop(0, e_per_w // B, block_body, 0)

    return k(src_idx, dst_idx, z)


def kernel(z, edge_index):
    idx = edge_index.astype(jnp.int32)
    return _run(idx[0], idx[1], z, idx.shape[1])
